# baseline (speedup 1.0000x reference)
import dataclasses

import jax
import jax.numpy as jnp
from jax import lax
from jax.experimental import pallas as pl
from jax.experimental.pallas import tpu as pltpu
from jax.experimental.pallas import tpu_sc as plsc

N_NODES = 10000
N_PAD = 10240
N_EDGES = 320000
D_IN = 128
D_OUT = 16

NUM_CORES = 2
NUM_SUBCORES = 16
NUM_TILES = NUM_CORES * NUM_SUBCORES
CHUNK = 2000
NUM_CHUNKS = 5
EDGES_PER_TILE = CHUNK * NUM_CHUNKS
ROWS_PER_TILE = N_PAD // NUM_SUBCORES
NBUF = 3


def _lane_bcast(vec, j):
    idx = jnp.full((16, 1), j, jnp.int32)
    dnums = lax.GatherDimensionNumbers(
        offset_dims=(), collapsed_slice_dims=(0,), start_index_map=(0,))
    return lax.gather(vec, idx, dnums, (1,),
                      mode=lax.GatherScatterMode.PROMISE_IN_BOUNDS)


def _matmul_body(x_ref, w_ref, o_ref):
    wp = jnp.concatenate(
        [w_ref[...], jnp.zeros((D_IN, 128 - D_OUT), jnp.float32)], axis=1)
    o_ref[...] = jnp.dot(x_ref[...], wp, preferred_element_type=jnp.float32)


def _tc_matmul(x, W):
    nblk = 10
    return pl.pallas_call(
        _matmul_body,
        grid=(nblk,),
        in_specs=[pl.BlockSpec((N_NODES // nblk, D_IN), lambda i: (i, 0)),
                  pl.BlockSpec((D_IN, D_OUT), lambda i: (0, 0))],
        out_specs=pl.BlockSpec((N_NODES // nblk, 128), lambda i: (i, 0)),
        out_shape=jax.ShapeDtypeStruct((N_NODES, 128), jnp.float32),
    )(x, W)


def _sc_body(xw_hbm, ei_hbm, w_hbm, out_hbm,
             src_v, dst_v, w_v, rows_v, bounce_v, agg_sh, sem_g, sem_s):
    cid = lax.axis_index("c")
    sid = lax.axis_index("s")
    wid = cid * NUM_SUBCORES + sid
    tile_base = wid * EDGES_PER_TILE

    def fetch_idx(k):
        b2 = k % 2
        base = tile_base + k * CHUNK
        pltpu.sync_copy(ei_hbm.at[0].at[pl.ds(base, CHUNK)], src_v.at[b2])
        pltpu.sync_copy(ei_hbm.at[1].at[pl.ds(base, CHUNK)],
                        dst_v.at[k % NBUF])
        pltpu.sync_copy(w_hbm.at[pl.ds(base, CHUNK)], w_v.at[b2])

        @plsc.parallel_loop(0, CHUNK, step=16, unroll=2)
        def _(i, _b2=b2):
            sl = pl.ds(i, 16)
            src_v.at[_b2].at[sl][...] = src_v.at[_b2].at[sl][...] * 8

    def start_gather(k):
        pltpu.async_copy(xw_hbm.at[src_v.at[k % 2]], rows_v.at[k % NBUF],
                         sem_g.at[k % NBUF])

    fetch_idx(0)
    start_gather(0)

    @pl.loop(0, ROWS_PER_TILE)
    def _(i):
        rows_v.at[2].at[i][...] = jnp.zeros((D_OUT,), jnp.float32)

    pltpu.sync_copy(rows_v.at[2].at[pl.ds(0, ROWS_PER_TILE)],
                    agg_sh.at[pl.ds(sid * ROWS_PER_TILE, ROWS_PER_TILE)])

    fetch_idx(1)
    start_gather(1)
    plsc.subcore_barrier()

    for k in range(NUM_CHUNKS):
        rb = k % NBUF
        pltpu.make_async_copy(xw_hbm.at[src_v.at[k % 2]], rows_v.at[rb],
                              sem_g.at[rb]).wait()

        @plsc.parallel_loop(0, CHUNK, step=16, unroll=2)
        def _(i0, _rb=rb, _b2=k % 2):
            wv = w_v.at[_b2].at[pl.ds(i0, 16)][...]
            for j in range(16):
                wb = _lane_bcast(wv, j)
                rows_v.at[_rb].at[i0 + j][...] = (
                    rows_v.at[_rb].at[i0 + j][...] * wb)

        pltpu.async_copy(rows_v.at[rb], agg_sh.at[dst_v.at[rb]],
                         sem_s.at[rb], add=True)

        if k + 2 < NUM_CHUNKS:
            if k - 1 >= 0:
                pb = (k - 1) % NBUF
                pltpu.make_async_copy(rows_v.at[pb],
                                      agg_sh.at[dst_v.at[pb]],
                                      sem_s.at[pb]).wait()
            fetch_idx(k + 2)
            start_gather(k + 2)

    for k in range(max(0, NUM_CHUNKS - NBUF), NUM_CHUNKS):
        pltpu.make_async_copy(rows_v.at[k % NBUF],
                              agg_sh.at[dst_v.at[k % NBUF]],
                              sem_s.at[k % NBUF]).wait()

    plsc.subcore_barrier()
    pltpu.sync_copy(agg_sh.at[pl.ds(sid * ROWS_PER_TILE, ROWS_PER_TILE)],
                    rows_v.at[0].at[pl.ds(0, ROWS_PER_TILE)])

    @pl.loop(0, ROWS_PER_TILE // 8)
    def _(r):
        for g in range(8):
            bounce_v.at[r, pl.ds(g * D_OUT, D_OUT)][...] = (
                rows_v.at[0].at[r * 8 + g][...])

    pltpu.sync_copy(bounce_v,
                    out_hbm.at[cid].at[pl.ds(sid * (ROWS_PER_TILE // 8),
                                             ROWS_PER_TILE // 8)])


def _sc_aggregate(xw, edge_index, w):
    mesh = plsc.VectorSubcoreMesh(core_axis_name="c", subcore_axis_name="s")
    cp = pltpu.CompilerParams()
    if "needs_layout_passes" in pltpu.CompilerParams.__dataclass_fields__:
        cp = dataclasses.replace(cp, needs_layout_passes=False)
    if "use_tc_tiling_on_sc" in pltpu.CompilerParams.__dataclass_fields__:
        cp = dataclasses.replace(cp, use_tc_tiling_on_sc=False)
    return pl.kernel(
        _sc_body,
        out_type=jax.ShapeDtypeStruct((NUM_CORES, N_PAD // 8, 8 * D_OUT),
                                      jnp.float32),
        mesh=mesh,
        scratch_types=[
            pltpu.VMEM((2, CHUNK), jnp.int32),
            pltpu.VMEM((NBUF, CHUNK), jnp.int32),
            pltpu.VMEM((2, CHUNK), jnp.float32),
            pltpu.VMEM((NBUF, CHUNK, D_OUT), jnp.float32),
            pltpu.VMEM((ROWS_PER_TILE // 8, 8 * D_OUT), jnp.float32),
            pltpu.VMEM_SHARED((N_PAD, D_OUT), jnp.float32),
            pltpu.SemaphoreType.DMA((NBUF,)),
            pltpu.SemaphoreType.DMA((NBUF,)),
        ],
        compiler_params=cp,
    )(xw, edge_index, w)


def _epilogue_body(p_ref, b_ref, m_ref, o_ref):
    bb = jnp.concatenate([b_ref[...]] * 8, axis=1)
    z = jnp.maximum(p_ref[0] + p_ref[1] + bb, 0.0)
    m = jnp.max(z, axis=1, keepdims=True)
    e = jnp.exp(z - m)
    s = jnp.dot(e, m_ref[...], preferred_element_type=jnp.float32)
    out = (z - m) - jnp.log(s)
    o_ref[...] = out[:N_NODES // 8]


def _tc_epilogue(p, b):
    g = jnp.arange(8 * D_OUT, dtype=jnp.int32) // D_OUT
    gmat = (g[:, None] == g[None, :]).astype(jnp.float32)
    return pl.pallas_call(
        _epilogue_body,
        out_shape=jax.ShapeDtypeStruct((N_NODES // 8, 8 * D_OUT),
                                       jnp.float32),
    )(p, b.reshape(1, D_OUT), gmat)


def kernel(x, edge_index, edge_weight, W, b):
    xw = _tc_matmul(x, W)
    xw16 = xw.reshape(8 * N_NODES, D_OUT)
    partials = _sc_aggregate(xw16, edge_index.astype(jnp.int32), edge_weight)
    out = _tc_epilogue(partials, b)
    return out.reshape(N_NODES, D_OUT)


# speedup vs baseline: 1.0602x; 1.0602x over previous
import dataclasses

import jax
import jax.numpy as jnp
from jax import lax
from jax.experimental import pallas as pl
from jax.experimental.pallas import tpu as pltpu
from jax.experimental.pallas import tpu_sc as plsc

N_NODES = 10000
N_PAD = 10240
N_EDGES = 320000
D_IN = 128
D_OUT = 16

NUM_CORES = 2
NUM_SUBCORES = 16
NUM_TILES = NUM_CORES * NUM_SUBCORES
CHUNK = 2000
NUM_CHUNKS = 5
EDGES_PER_TILE = CHUNK * NUM_CHUNKS
ROWS_PER_TILE = N_PAD // NUM_SUBCORES
NBUF = 3


def _lane_bcast(vec, j):
    idx = jnp.full((16, 1), j, jnp.int32)
    dnums = lax.GatherDimensionNumbers(
        offset_dims=(), collapsed_slice_dims=(0,), start_index_map=(0,))
    return lax.gather(vec, idx, dnums, (1,),
                      mode=lax.GatherScatterMode.PROMISE_IN_BOUNDS)


def _matmul_body(x_ref, w_ref, o_ref):
    wp = jnp.concatenate(
        [w_ref[...], jnp.zeros((D_IN, 128 - D_OUT), jnp.float32)], axis=1)
    o_ref[...] = jnp.dot(x_ref[...], wp, preferred_element_type=jnp.float32)


def _tc_matmul(x, W):
    return pl.pallas_call(
        _matmul_body,
        out_shape=jax.ShapeDtypeStruct((N_NODES, 128), jnp.float32),
    )(x, W)


def _sc_body(xw_hbm, ei_hbm, w_hbm, out_hbm,
             src_v, dst_v, w_v, rows_v, bounce_v, agg_sh, sem_g, sem_s):
    cid = lax.axis_index("c")
    sid = lax.axis_index("s")
    wid = cid * NUM_SUBCORES + sid
    tile_base = wid * EDGES_PER_TILE

    def fetch_idx(k):
        b2 = k % 2
        base = tile_base + k * CHUNK
        pltpu.sync_copy(ei_hbm.at[0].at[pl.ds(base, CHUNK)], src_v.at[b2])
        pltpu.sync_copy(ei_hbm.at[1].at[pl.ds(base, CHUNK)],
                        dst_v.at[k % NBUF])
        pltpu.sync_copy(w_hbm.at[pl.ds(base, CHUNK)], w_v.at[b2])

        @plsc.parallel_loop(0, CHUNK, step=16, unroll=2)
        def _(i, _b2=b2):
            sl = pl.ds(i, 16)
            src_v.at[_b2].at[sl][...] = src_v.at[_b2].at[sl][...] * 8

    def start_gather(k):
        pltpu.async_copy(xw_hbm.at[src_v.at[k % 2]], rows_v.at[k % NBUF],
                         sem_g.at[k % NBUF])

    fetch_idx(0)
    start_gather(0)

    @pl.loop(0, ROWS_PER_TILE)
    def _(i):
        rows_v.at[2].at[i][...] = jnp.zeros((D_OUT,), jnp.float32)

    pltpu.sync_copy(rows_v.at[2].at[pl.ds(0, ROWS_PER_TILE)],
                    agg_sh.at[pl.ds(sid * ROWS_PER_TILE, ROWS_PER_TILE)])

    fetch_idx(1)
    start_gather(1)
    plsc.subcore_barrier()

    for k in range(NUM_CHUNKS):
        rb = k % NBUF
        pltpu.make_async_copy(xw_hbm.at[src_v.at[k % 2]], rows_v.at[rb],
                              sem_g.at[rb]).wait()

        @plsc.parallel_loop(0, CHUNK, step=16, unroll=2)
        def _(i0, _rb=rb, _b2=k % 2):
            wv = w_v.at[_b2].at[pl.ds(i0, 16)][...]
            for j in range(16):
                wb = _lane_bcast(wv, j)
                rows_v.at[_rb].at[i0 + j][...] = (
                    rows_v.at[_rb].at[i0 + j][...] * wb)

        pltpu.async_copy(rows_v.at[rb], agg_sh.at[dst_v.at[rb]],
                         sem_s.at[rb], add=True)

        if k + 2 < NUM_CHUNKS:
            if k - 1 >= 0:
                pb = (k - 1) % NBUF
                pltpu.make_async_copy(rows_v.at[pb],
                                      agg_sh.at[dst_v.at[pb]],
                                      sem_s.at[pb]).wait()
            fetch_idx(k + 2)
            start_gather(k + 2)

    for k in range(max(0, NUM_CHUNKS - NBUF), NUM_CHUNKS):
        pltpu.make_async_copy(rows_v.at[k % NBUF],
                              agg_sh.at[dst_v.at[k % NBUF]],
                              sem_s.at[k % NBUF]).wait()

    plsc.subcore_barrier()
    pltpu.sync_copy(agg_sh.at[pl.ds(sid * ROWS_PER_TILE, ROWS_PER_TILE)],
                    rows_v.at[0].at[pl.ds(0, ROWS_PER_TILE)])

    @pl.loop(0, ROWS_PER_TILE // 8)
    def _(r):
        for g in range(8):
            bounce_v.at[r, pl.ds(g * D_OUT, D_OUT)][...] = (
                rows_v.at[0].at[r * 8 + g][...])

    pltpu.sync_copy(bounce_v,
                    out_hbm.at[cid].at[pl.ds(sid * (ROWS_PER_TILE // 8),
                                             ROWS_PER_TILE // 8)])


def _sc_aggregate(xw, edge_index, w):
    mesh = plsc.VectorSubcoreMesh(core_axis_name="c", subcore_axis_name="s")
    cp = pltpu.CompilerParams()
    if "needs_layout_passes" in pltpu.CompilerParams.__dataclass_fields__:
        cp = dataclasses.replace(cp, needs_layout_passes=False)
    if "use_tc_tiling_on_sc" in pltpu.CompilerParams.__dataclass_fields__:
        cp = dataclasses.replace(cp, use_tc_tiling_on_sc=False)
    return pl.kernel(
        _sc_body,
        out_type=jax.ShapeDtypeStruct((NUM_CORES, N_PAD // 8, 8 * D_OUT),
                                      jnp.float32),
        mesh=mesh,
        scratch_types=[
            pltpu.VMEM((2, CHUNK), jnp.int32),
            pltpu.VMEM((NBUF, CHUNK), jnp.int32),
            pltpu.VMEM((2, CHUNK), jnp.float32),
            pltpu.VMEM((NBUF, CHUNK, D_OUT), jnp.float32),
            pltpu.VMEM((ROWS_PER_TILE // 8, 8 * D_OUT), jnp.float32),
            pltpu.VMEM_SHARED((N_PAD, D_OUT), jnp.float32),
            pltpu.SemaphoreType.DMA((NBUF,)),
            pltpu.SemaphoreType.DMA((NBUF,)),
        ],
        compiler_params=cp,
    )(xw, edge_index, w)


def _epilogue_body(p_ref, b_ref, m_ref, o_ref):
    bb = jnp.concatenate([b_ref[...]] * 8, axis=1)
    z = jnp.maximum(p_ref[0] + p_ref[1] + bb, 0.0)
    m = jnp.max(z, axis=1, keepdims=True)
    e = jnp.exp(z - m)
    s = jnp.dot(e, m_ref[...], preferred_element_type=jnp.float32)
    out = (z - m) - jnp.log(s)
    o_ref[...] = out[:N_NODES // 8]


def _tc_epilogue(p, b):
    g = jnp.arange(8 * D_OUT, dtype=jnp.int32) // D_OUT
    gmat = (g[:, None] == g[None, :]).astype(jnp.float32)
    return pl.pallas_call(
        _epilogue_body,
        out_shape=jax.ShapeDtypeStruct((N_NODES // 8, 8 * D_OUT),
                                       jnp.float32),
    )(p, b.reshape(1, D_OUT), gmat)


def kernel(x, edge_index, edge_weight, W, b):
    xw = _tc_matmul(x, W)
    xw16 = xw.reshape(8 * N_NODES, D_OUT)
    partials = _sc_aggregate(xw16, edge_index.astype(jnp.int32), edge_weight)
    out = _tc_epilogue(partials, b)
    return out.reshape(N_NODES, D_OUT)


# speedup vs baseline: 1.0637x; 1.0033x over previous
import dataclasses

import jax
import jax.numpy as jnp
from jax import lax
from jax.experimental import pallas as pl
from jax.experimental.pallas import tpu as pltpu
from jax.experimental.pallas import tpu_sc as plsc

N_NODES = 10000
N_PAD = 10240
N_EDGES = 320000
D_IN = 128
D_OUT = 16

NUM_CORES = 2
NUM_SUBCORES = 16
NUM_TILES = NUM_CORES * NUM_SUBCORES
CHUNK = 2000
NUM_CHUNKS = 5
EDGES_PER_TILE = CHUNK * NUM_CHUNKS
ROWS_PER_TILE = N_PAD // NUM_SUBCORES
NBUF = 3


def _lane_bcast(vec, j):
    idx = jnp.full((16, 1), j, jnp.int32)
    dnums = lax.GatherDimensionNumbers(
        offset_dims=(), collapsed_slice_dims=(0,), start_index_map=(0,))
    return lax.gather(vec, idx, dnums, (1,),
                      mode=lax.GatherScatterMode.PROMISE_IN_BOUNDS)


def _matmul_body(x_ref, w_ref, o_ref):
    wp = jnp.concatenate(
        [w_ref[...], jnp.zeros((D_IN, 128 - D_OUT), jnp.float32)], axis=1)
    o_ref[...] = jnp.dot(x_ref[...], wp, preferred_element_type=jnp.float32)


def _tc_matmul(x, W):
    return pl.pallas_call(
        _matmul_body,
        out_shape=jax.ShapeDtypeStruct((N_NODES, 128), jnp.float32),
    )(x, W)


def _sc_body(xw_hbm, ei_hbm, w_hbm, out_hbm,
             src_v, dst_v, w_v, rows_v, bounce_v, agg_sh, sem_g, sem_s):
    cid = lax.axis_index("c")
    sid = lax.axis_index("s")
    wid = cid * NUM_SUBCORES + sid
    tile_base = wid * EDGES_PER_TILE

    def fetch_idx(k):
        b2 = k % 2
        base = tile_base + k * CHUNK
        pltpu.sync_copy(ei_hbm.at[0].at[pl.ds(base, CHUNK)], src_v.at[b2])
        pltpu.sync_copy(ei_hbm.at[1].at[pl.ds(base, CHUNK)],
                        dst_v.at[k % NBUF])
        pltpu.sync_copy(w_hbm.at[pl.ds(base, CHUNK)], w_v.at[b2])

    def start_gather(k):
        pltpu.async_copy(xw_hbm.at[src_v.at[k % 2]], rows_v.at[k % NBUF],
                         sem_g.at[k % NBUF])

    fetch_idx(0)
    start_gather(0)

    @pl.loop(0, ROWS_PER_TILE)
    def _(i):
        rows_v.at[2].at[i][...] = jnp.zeros((D_OUT,), jnp.float32)

    pltpu.sync_copy(rows_v.at[2].at[pl.ds(0, ROWS_PER_TILE)],
                    agg_sh.at[pl.ds(sid * ROWS_PER_TILE, ROWS_PER_TILE)])

    fetch_idx(1)
    start_gather(1)
    plsc.subcore_barrier()

    for k in range(NUM_CHUNKS):
        rb = k % NBUF
        pltpu.make_async_copy(xw_hbm.at[src_v.at[k % 2]], rows_v.at[rb],
                              sem_g.at[rb]).wait()

        @plsc.parallel_loop(0, CHUNK, step=16, unroll=2)
        def _(i0, _rb=rb, _b2=k % 2):
            wv = w_v.at[_b2].at[pl.ds(i0, 16)][...]
            for j in range(16):
                wb = _lane_bcast(wv, j)
                rows_v.at[_rb].at[i0 + j][...] = (
                    rows_v.at[_rb].at[i0 + j][...] * wb)

        pltpu.async_copy(rows_v.at[rb], agg_sh.at[dst_v.at[rb]],
                         sem_s.at[rb], add=True)

        if k + 2 < NUM_CHUNKS:
            if k - 1 >= 0:
                pb = (k - 1) % NBUF
                pltpu.make_async_copy(rows_v.at[pb],
                                      agg_sh.at[dst_v.at[pb]],
                                      sem_s.at[pb]).wait()
            fetch_idx(k + 2)
            start_gather(k + 2)

    for k in range(max(0, NUM_CHUNKS - NBUF), NUM_CHUNKS):
        pltpu.make_async_copy(rows_v.at[k % NBUF],
                              agg_sh.at[dst_v.at[k % NBUF]],
                              sem_s.at[k % NBUF]).wait()

    plsc.subcore_barrier()
    pltpu.sync_copy(agg_sh.at[pl.ds(sid * ROWS_PER_TILE, ROWS_PER_TILE)],
                    rows_v.at[0].at[pl.ds(0, ROWS_PER_TILE)])

    @pl.loop(0, ROWS_PER_TILE // 8)
    def _(r):
        for g in range(8):
            bounce_v.at[r, pl.ds(g * D_OUT, D_OUT)][...] = (
                rows_v.at[0].at[r * 8 + g][...])

    pltpu.sync_copy(bounce_v,
                    out_hbm.at[cid].at[pl.ds(sid * (ROWS_PER_TILE // 8),
                                             ROWS_PER_TILE // 8)])


def _sc_aggregate(xw, edge_index, w):
    mesh = plsc.VectorSubcoreMesh(core_axis_name="c", subcore_axis_name="s")
    cp = pltpu.CompilerParams()
    if "needs_layout_passes" in pltpu.CompilerParams.__dataclass_fields__:
        cp = dataclasses.replace(cp, needs_layout_passes=False)
    if "use_tc_tiling_on_sc" in pltpu.CompilerParams.__dataclass_fields__:
        cp = dataclasses.replace(cp, use_tc_tiling_on_sc=False)
    return pl.kernel(
        _sc_body,
        out_type=jax.ShapeDtypeStruct((NUM_CORES, N_PAD // 8, 8 * D_OUT),
                                      jnp.float32),
        mesh=mesh,
        scratch_types=[
            pltpu.VMEM((2, CHUNK), jnp.int32),
            pltpu.VMEM((NBUF, CHUNK), jnp.int32),
            pltpu.VMEM((2, CHUNK), jnp.float32),
            pltpu.VMEM((NBUF, CHUNK, D_OUT), jnp.float32),
            pltpu.VMEM((ROWS_PER_TILE // 8, 8 * D_OUT), jnp.float32),
            pltpu.VMEM_SHARED((N_PAD, D_OUT), jnp.float32),
            pltpu.SemaphoreType.DMA((NBUF,)),
            pltpu.SemaphoreType.DMA((NBUF,)),
        ],
        compiler_params=cp,
    )(xw, edge_index, w)


def _epilogue_body(p_ref, b_ref, m_ref, o_ref):
    bb = jnp.concatenate([b_ref[...]] * 8, axis=1)
    z = jnp.maximum(p_ref[0] + p_ref[1] + bb, 0.0)
    m = jnp.max(z, axis=1, keepdims=True)
    e = jnp.exp(z - m)
    s = jnp.dot(e, m_ref[...], preferred_element_type=jnp.float32)
    out = (z - m) - jnp.log(s)
    o_ref[...] = out[:N_NODES // 8]


def _tc_epilogue(p, b):
    g = jnp.arange(8 * D_OUT, dtype=jnp.int32) // D_OUT
    gmat = (g[:, None] == g[None, :]).astype(jnp.float32)
    return pl.pallas_call(
        _epilogue_body,
        out_shape=jax.ShapeDtypeStruct((N_NODES // 8, 8 * D_OUT),
                                       jnp.float32),
    )(p, b.reshape(1, D_OUT), gmat)


def kernel(x, edge_index, edge_weight, W, b):
    xw = _tc_matmul(x, W)
    xw16 = xw.reshape(8 * N_NODES, D_OUT)
    ei8 = edge_index.astype(jnp.int32) * jnp.array([[8], [1]], jnp.int32)
    partials = _sc_aggregate(xw16, ei8, edge_weight)
    out = _tc_epilogue(partials, b)
    return out.reshape(N_NODES, D_OUT)
